# baseline (speedup 1.0000x reference)
import jax
import jax.numpy as jnp
from jax import lax
from jax.experimental import pallas as pl
from jax.experimental.pallas import tpu as pltpu
from jax.experimental.pallas import tpu_sc as plsc

NUM_CORES = 2
NUM_SUBCORES = 16
LANES = 16
NW = NUM_CORES * NUM_SUBCORES

EMBED = 64
BATCH = 16384
ROWS_PER_W = BATCH // NW
CHUNK = 128
NCHUNK = ROWS_PER_W // CHUNK
KSUB = CHUNK // LANES
PAIR = 2 * EMBED


def _dot_kernel(urow_hbm, mrow_hbm, uoff_hbm, moff_hbm,
                utab_hbm, mtab_hbm, out_hbm,
                uidx_v, midx_v, uoff_v, moff_v, ubuf_v, mbuf_v, out_v, sem):
    wid = lax.axis_index("s") * NUM_CORES + lax.axis_index("c")
    base = wid * ROWS_PER_W

    pltpu.sync_copy(urow_hbm.at[wid], uidx_v)
    pltpu.sync_copy(mrow_hbm.at[wid], midx_v)
    pltpu.sync_copy(uoff_hbm.at[wid], uoff_v)
    pltpu.sync_copy(moff_hbm.at[wid], moff_v)

    iota = lax.iota(jnp.int32, LANES)

    @pl.loop(0, NCHUNK)
    def _(c):
        cu = pltpu.async_copy(utab_hbm.at[uidx_v.at[c]], ubuf_v, sem)
        cm = pltpu.async_copy(mtab_hbm.at[midx_v.at[c]], mbuf_v, sem)
        cu.wait()
        cm.wait()

        @pl.loop(0, KSUB)
        def _(k):
            rows = k * LANES + iota
            ucol0 = uoff_v[c, pl.ds(k * LANES, LANES)]
            mcol0 = moff_v[c, pl.ds(k * LANES, LANES)]
            acc = jnp.zeros((LANES,), jnp.float32)
            for j in range(EMBED):
                u = plsc.load_gather(ubuf_v, [rows, ucol0 + j])
                m = plsc.load_gather(mbuf_v, [rows, mcol0 + j])
                acc = acc + u * m
            out_v[pl.ds(c * CHUNK + k * LANES, LANES)] = acc

    pltpu.sync_copy(out_v, out_hbm.at[pl.ds(base, ROWS_PER_W)])


@jax.jit
def _run(user_ids, movie_ids, user_table, movie_table):
    mesh = plsc.VectorSubcoreMesh(core_axis_name="c", subcore_axis_name="s",
                                  num_cores=NUM_CORES,
                                  num_subcores=NUM_SUBCORES)
    cp = pltpu.CompilerParams(needs_layout_passes=False,
                              use_tc_tiling_on_sc=True)
    kern = pl.kernel(
        _dot_kernel,
        out_type=jax.ShapeDtypeStruct((BATCH,), jnp.float32),
        mesh=mesh,
        compiler_params=cp,
        scratch_types=[
            pltpu.VMEM((NCHUNK, CHUNK), jnp.int32),
            pltpu.VMEM((NCHUNK, CHUNK), jnp.int32),
            pltpu.VMEM((NCHUNK, CHUNK), jnp.int32),
            pltpu.VMEM((NCHUNK, CHUNK), jnp.int32),
            pltpu.VMEM((CHUNK, PAIR), jnp.float32),
            pltpu.VMEM((CHUNK, PAIR), jnp.float32),
            pltpu.VMEM((ROWS_PER_W,), jnp.float32),
            pltpu.SemaphoreType.DMA,
        ],
    )
    uids = user_ids.astype(jnp.int32)
    mids = movie_ids.astype(jnp.int32)
    urow = (uids >> 1).reshape(NW, NCHUNK, CHUNK)
    mrow = (mids >> 1).reshape(NW, NCHUNK, CHUNK)
    uoff = ((uids & 1) * EMBED).reshape(NW, NCHUNK, CHUNK)
    moff = ((mids & 1) * EMBED).reshape(NW, NCHUNK, CHUNK)
    utab = user_table.reshape(-1, PAIR)
    mtab = movie_table.reshape(-1, PAIR)
    return kern(urow, mrow, uoff, moff, utab, mtab)


def kernel(user_ids, movie_ids, user_table, movie_table):
    out = _run(user_ids, movie_ids, user_table, movie_table)
    return out.reshape(BATCH, 1)
